# baseline (speedup 1.0000x reference)
import functools

import jax
import jax.numpy as jnp
from jax import lax
from jax.experimental import pallas as pl
from jax.experimental.pallas import tpu as pltpu
from jax.experimental.pallas import tpu_sc as plsc

B, N, V, DIN, DOUT = 4096, 50, 1000000, 64, 64

_NC, _NS = 2, 16
_NW = _NC * _NS
_ROWS = B * N
_PER_W = _ROWS // _NW
_CHUNK = 128
_NCHUNK = _PER_W // _CHUNK

@functools.cache
def _make_sc_gather():
    mesh = plsc.VectorSubcoreMesh(core_axis_name="c", subcore_axis_name="s")

    @functools.partial(
        pl.kernel,
        mesh=mesh,
        compiler_params=pltpu.CompilerParams(use_tc_tiling_on_sc=False),
        out_type=jax.ShapeDtypeStruct((_ROWS, DIN), jnp.float32),
        scratch_types=[
            pltpu.VMEM((_NCHUNK, _CHUNK), jnp.int32),
            pltpu.VMEM((_CHUNK, DIN), jnp.float32),
            pltpu.SemaphoreType.DMA,
        ],
    )
    def _sc_gather(idx_hbm, table_hbm, out_hbm, idx_v, rows_v, sem):
        wid = lax.axis_index("s") * _NC + lax.axis_index("c")
        pltpu.sync_copy(idx_hbm.at[wid], idx_v)
        base = wid * _PER_W

        def body(j, carry):
            pltpu.async_copy(table_hbm.at[idx_v.at[j]], rows_v, sem).wait()
            pltpu.sync_copy(
                rows_v, out_hbm.at[pl.ds(base + j * _CHUNK, _CHUNK)]
            )
            return carry

        lax.fori_loop(0, _NCHUNK, body, 0)

    return _sc_gather


_BB = 256


def _tc_body(e_ref, sq_ref, w1_ref, b1_ref, w2_ref, b2_ref, o_ref):
    w1 = w1_ref[...]
    b1 = b1_ref[...]
    acc = jnp.zeros((_BB, DOUT), jnp.float32)
    for n in range(N):
        e = e_ref[:, n, :]
        h = jnp.dot(e, w1, preferred_element_type=jnp.float32) + b1
        acc = acc + jnp.maximum(h, 0.0)
    pooled = acc / sq_ref[...]
    o_ref[...] = (
        jnp.dot(pooled, w2_ref[...], preferred_element_type=jnp.float32)
        + b2_ref[...]
    )


def _tc_mlp(e3, sq2, W1, b1, W2, b2):
    grid = (B // _BB,)
    return pl.pallas_call(
        _tc_body,
        grid=grid,
        in_specs=[
            pl.BlockSpec((_BB, N, DIN), lambda i: (i, 0, 0)),
            pl.BlockSpec((_BB, 1), lambda i: (i, 0)),
            pl.BlockSpec((DIN, DOUT), lambda i: (0, 0)),
            pl.BlockSpec((1, DOUT), lambda i: (0, 0)),
            pl.BlockSpec((DOUT, DOUT), lambda i: (0, 0)),
            pl.BlockSpec((1, DOUT), lambda i: (0, 0)),
        ],
        out_specs=pl.BlockSpec((_BB, DOUT), lambda i: (i, 0)),
        out_shape=jax.ShapeDtypeStruct((B, DOUT), jnp.float32),
    )(e3, sq2, W1, b1, W2, b2)


def kernel(x, sq_lengths, weight, W1, b1, W2, b2):
    idx3d = x.reshape(_NW, _NCHUNK, _CHUNK)
    e = _make_sc_gather()(idx3d, weight)
    e3 = e.reshape(B, N, DIN)
    return _tc_mlp(
        e3,
        sq_lengths.reshape(B, 1),
        W1,
        b1.reshape(1, DOUT),
        W2,
        b2.reshape(1, DOUT),
    )


# speedup vs baseline: 1.2025x; 1.2025x over previous
import functools

import jax
import jax.numpy as jnp
from jax import lax
from jax.experimental import pallas as pl
from jax.experimental.pallas import tpu as pltpu
from jax.experimental.pallas import tpu_sc as plsc

B, N, V, DIN, DOUT = 4096, 50, 1000000, 64, 64
_EW = 128

_NC, _NS = 2, 16
_NW = _NC * _NS
_ROWS = B * N
_PER_W = _ROWS // _NW
_CHUNK = 128
_NCHUNK = _PER_W // _CHUNK


@functools.cache
def _make_sc_gather():
    mesh = plsc.VectorSubcoreMesh(core_axis_name="c", subcore_axis_name="s")

    @functools.partial(
        pl.kernel,
        mesh=mesh,
        compiler_params=pltpu.CompilerParams(use_tc_tiling_on_sc=False),
        out_type=jax.ShapeDtypeStruct((_ROWS, _EW), jnp.float32),
        scratch_types=[
            pltpu.VMEM((_NCHUNK, _CHUNK), jnp.int32),
            pltpu.VMEM((2, _CHUNK, DIN), jnp.float32),
            pltpu.SemaphoreType.DMA,
            pltpu.SemaphoreType.DMA,
        ],
    )
    def _sc_gather(idx_hbm, table_hbm, out_hbm, idx_v, rows_v, sem0, sem1):
        wid = lax.axis_index("s") * _NC + lax.axis_index("c")
        pltpu.sync_copy(idx_hbm.at[wid], idx_v)
        base = wid * _PER_W
        sems = (sem0, sem1)

        def start(j, slot):
            pltpu.async_copy(
                table_hbm.at[idx_v.at[j]], rows_v.at[slot], sems[slot]
            )

        start(0, 0)
        start(1, 1)

        def body(g, carry):
            for slot in range(2):
                j = 2 * g + slot
                pltpu.make_async_copy(
                    table_hbm.at[idx_v.at[j]], rows_v.at[slot], sems[slot]
                ).wait()
                pltpu.sync_copy(
                    rows_v.at[slot],
                    out_hbm.at[
                        pl.ds(base + j * _CHUNK, _CHUNK), pl.ds(0, DIN)
                    ],
                )

                @pl.when(j + 2 < _NCHUNK)
                def _():
                    start(j + 2, slot)

            return carry

        lax.fori_loop(0, _NCHUNK // 2, body, 0)

    return _sc_gather


_BB = 256


def _tc_body(e_ref, sq_ref, w1_ref, b1_ref, w2_ref, b2_ref, o_ref):
    w1 = w1_ref[...]
    b1 = b1_ref[...]
    acc = jnp.zeros((_BB, DOUT), jnp.float32)
    for n in range(N):
        e = e_ref[n][:, :DIN]
        h = jnp.dot(e, w1, preferred_element_type=jnp.float32) + b1
        acc = acc + jnp.maximum(h, 0.0)
    pooled = acc / sq_ref[...]
    o_ref[...] = (
        jnp.dot(pooled, w2_ref[...], preferred_element_type=jnp.float32)
        + b2_ref[...]
    )


def _tc_mlp(e3, sq2, W1, b1, W2, b2):
    grid = (B // _BB,)
    return pl.pallas_call(
        _tc_body,
        grid=grid,
        in_specs=[
            pl.BlockSpec((N, _BB, _EW), lambda i: (0, i, 0)),
            pl.BlockSpec((_BB, 1), lambda i: (i, 0)),
            pl.BlockSpec((DIN, DOUT), lambda i: (0, 0)),
            pl.BlockSpec((1, DOUT), lambda i: (0, 0)),
            pl.BlockSpec((DOUT, DOUT), lambda i: (0, 0)),
            pl.BlockSpec((1, DOUT), lambda i: (0, 0)),
        ],
        out_specs=pl.BlockSpec((_BB, DOUT), lambda i: (i, 0)),
        out_shape=jax.ShapeDtypeStruct((B, DOUT), jnp.float32),
    )(e3, sq2, W1, b1, W2, b2)


def kernel(x, sq_lengths, weight, W1, b1, W2, b2):
    idx3d = x.T.reshape(_NW, _NCHUNK, _CHUNK)
    e = _make_sc_gather()(idx3d, weight)
    e3 = e.reshape(N, B, _EW)
    return _tc_mlp(
        e3,
        sq_lengths.reshape(B, 1),
        W1,
        b1.reshape(1, DOUT),
        W2,
        b2.reshape(1, DOUT),
    )
